# baseline (speedup 1.0000x reference)
import functools

import jax
import jax.numpy as jnp
from jax import lax
from jax.experimental import pallas as pl
from jax.experimental.pallas import tpu as pltpu
from jax.experimental.pallas import tpu_sc as plsc

_B, _H, _D = 4096, 200, 64
_NC, _NS = 2, 16
_NW = _NC * _NS
_BPW = _B // _NW


def _emb_lookup(table, xT):
    mesh = plsc.VectorSubcoreMesh(core_axis_name="c", subcore_axis_name="s")

    @functools.partial(
        pl.kernel,
        mesh=mesh,
        out_type=jax.ShapeDtypeStruct((_H, _D // 8, _B // 128, 8 * 128),
                                      jnp.float32),
        compiler_params=pltpu.CompilerParams(use_tc_tiling_on_sc=False,
                                             needs_layout_passes=False),
        scratch_types=[
            pltpu.VMEM((_H, _BPW), jnp.int32),
            pltpu.VMEM((_BPW, _D), jnp.float32),
            pltpu.VMEM((_BPW, _D), jnp.float32),
            pltpu.VMEM((_D * _BPW,), jnp.float32),
            pltpu.VMEM((_D * _BPW,), jnp.float32),
            pltpu.SemaphoreType.DMA,
            pltpu.SemaphoreType.DMA,
            pltpu.SemaphoreType.DMA,
            pltpu.SemaphoreType.DMA,
        ],
    )
    def k(table_hbm, xT_hbm, out_hbm, idx_all, rows0, rows1, til0, til1,
          gs0, gs1, ss0, ss1):
        rows = (rows0, rows1)
        tils = (til0, til1)
        gsems = (gs0, gs1)
        ssems = (ss0, ss1)

        wid = lax.axis_index("s") * _NC + lax.axis_index("c")
        b0 = wid * _BPW

        def gather_start(h, b):
            pltpu.async_copy(table_hbm.at[idx_all.at[h]], rows[b], gsems[b])

        def gather_wait(h, b):
            pltpu.make_async_copy(table_hbm.at[idx_all.at[h]], rows[b],
                                  gsems[b]).wait()

        def store_start(h, b):
            for r in range(8):
                pltpu.async_copy(tils[b].at[pl.ds(r * 1024, 1024)],
                                 out_hbm.at[h, r, wid], ssems[b])

        def store_wait(b):
            for r in range(8):
                pltpu.make_async_copy(tils[b].at[pl.ds(r * 1024, 1024)],
                                      out_hbm.at[0, r, wid], ssems[b]).wait()

        lanes = lax.iota(jnp.int32, 16)
        base_vecs = [(lanes + kk * 16) * 128 for kk in range(_D // 16)]

        def transpose(b):
            def per_i(i, carry):
                for kk in range(_D // 16):
                    v = rows[b][i, pl.ds(kk * 16, 16)]
                    plsc.store_scatter(tils[b], [base_vecs[kk] + i], v)
                return carry

            lax.fori_loop(0, _BPW, per_i, 0)

        pltpu.sync_copy(xT_hbm.at[:, pl.ds(b0, _BPW)], idx_all)

        gather_start(0, 0)
        gather_wait(0, 0)
        gather_start(1, 1)
        transpose(0)
        store_start(0, 0)
        gather_wait(1, 1)
        gather_start(2, 0)
        transpose(1)
        store_start(1, 1)

        def group(g, carry):
            for b in range(2):
                h = g * 2 + b
                gather_wait(h, b)

                @pl.when(h + 1 < _H)
                def _():
                    gather_start(h + 1, (b + 1) % 2)

                store_wait(b)
                transpose(b)
                store_start(h, b)
            return carry

        lax.fori_loop(1, _H // 2, group, 0)

        store_wait(0)
        store_wait(1)

    return k(table, xT)


def kernel(x, table):
    xT = jnp.transpose(x)
    o = _emb_lookup(table, xT)
    o = o.reshape(_H, _D // 8, _B // 128, 8, 128)
    return jnp.transpose(o, (2, 4, 0, 1, 3)).reshape(_B, _H, _D)


# speedup vs baseline: 1.6173x; 1.6173x over previous
import functools

import jax
import jax.numpy as jnp
from jax import lax
from jax.experimental import pallas as pl
from jax.experimental.pallas import tpu as pltpu
from jax.experimental.pallas import tpu_sc as plsc

_B, _H, _D = 4096, 200, 64
_NC, _NS = 2, 16
_NW = _NC * _NS
_BPW = _B // _NW


def _emb_lookup(table, xT):
    mesh = plsc.VectorSubcoreMesh(core_axis_name="c", subcore_axis_name="s")

    @functools.partial(
        pl.kernel,
        mesh=mesh,
        out_type=jax.ShapeDtypeStruct((_H, _D // 8, _B // 128, 8, 128),
                                      jnp.float32),
        compiler_params=pltpu.CompilerParams(use_tc_tiling_on_sc=False,
                                             needs_layout_passes=False),
        scratch_types=[
            pltpu.VMEM((_H, _BPW), jnp.int32),
            pltpu.VMEM((_BPW, _D), jnp.float32),
            pltpu.VMEM((_BPW, _D), jnp.float32),
            pltpu.VMEM((_D, _BPW + 1), jnp.float32),
            pltpu.VMEM((_D, _BPW + 1), jnp.float32),
            pltpu.SemaphoreType.DMA,
            pltpu.SemaphoreType.DMA,
            pltpu.SemaphoreType.DMA,
            pltpu.SemaphoreType.DMA,
        ],
    )
    def k(table_hbm, xT_hbm, out_hbm, idx_all, rows0, rows1, til0, til1,
          gs0, gs1, ss0, ss1):
        rows = (rows0, rows1)
        tils = (til0, til1)
        gsems = (gs0, gs1)
        ssems = (ss0, ss1)

        wid = lax.axis_index("s") * _NC + lax.axis_index("c")
        b0 = wid * _BPW

        def gather_start(h, b):
            pltpu.async_copy(table_hbm.at[idx_all.at[h]], rows[b], gsems[b])

        def gather_wait(h, b):
            pltpu.make_async_copy(table_hbm.at[idx_all.at[h]], rows[b],
                                  gsems[b]).wait()

        def store_start(h, b):
            for r in range(8):
                pltpu.async_copy(tils[b].at[pl.ds(r * 8, 8), pl.ds(0, _BPW)],
                                 out_hbm.at[h, r, wid], ssems[b])

        def store_wait(b):
            for r in range(8):
                pltpu.make_async_copy(tils[b].at[pl.ds(r * 8, 8),
                                                 pl.ds(0, _BPW)],
                                      out_hbm.at[0, r, wid], ssems[b]).wait()

        lanes = lax.iota(jnp.int32, 16)
        dvecs = [lanes + kk * 16 for kk in range(_D // 16)]

        def transpose(b):
            def per_i(i, carry):
                for u in range(8):
                    ii = i * 8 + u
                    ivec = jnp.full((16,), ii, jnp.int32)
                    for kk in range(_D // 16):
                        v = rows[b][ii, pl.ds(kk * 16, 16)]
                        plsc.store_scatter(tils[b], [dvecs[kk], ivec], v)
                return carry

            lax.fori_loop(0, _BPW // 8, per_i, 0)

        pltpu.sync_copy(xT_hbm.at[:, pl.ds(b0, _BPW)], idx_all)

        gather_start(0, 0)
        gather_wait(0, 0)
        gather_start(1, 1)
        transpose(0)
        store_start(0, 0)
        gather_wait(1, 1)
        gather_start(2, 0)
        transpose(1)
        store_start(1, 1)

        def group(g, carry):
            for b in range(2):
                h = g * 2 + b
                gather_wait(h, b)

                @pl.when(h + 1 < _H)
                def _():
                    gather_start(h + 1, (b + 1) % 2)

                store_wait(b)
                transpose(b)
                store_start(h, b)
            return carry

        lax.fori_loop(1, _H // 2, group, 0)

        store_wait(0)
        store_wait(1)

    return k(table, xT)


def kernel(x, table):
    xT = jnp.transpose(x)
    o = _emb_lookup(table, xT)
    return jnp.transpose(o, (2, 4, 0, 1, 3)).reshape(_B, _H, _D)


# speedup vs baseline: 1.8855x; 1.1658x over previous
import functools

import jax
import jax.numpy as jnp
from jax import lax
from jax.experimental import pallas as pl
from jax.experimental.pallas import tpu as pltpu
from jax.experimental.pallas import tpu_sc as plsc

_B, _H, _D = 4096, 200, 64
_NC, _NS = 2, 16
_NW = _NC * _NS
_BPW = _B // _NW


def _emb_lookup(table, xT):
    mesh = plsc.VectorSubcoreMesh(core_axis_name="c", subcore_axis_name="s")

    @functools.partial(
        pl.kernel,
        mesh=mesh,
        out_type=jax.ShapeDtypeStruct((_H, _D // 8, _B // 128, 8, 128),
                                      jnp.float32),
        compiler_params=pltpu.CompilerParams(use_tc_tiling_on_sc=False,
                                             needs_layout_passes=False),
        scratch_types=[
            pltpu.VMEM((_H, _BPW), jnp.int32),
            pltpu.VMEM((_BPW, _D), jnp.float32),
            pltpu.VMEM((_BPW, _D), jnp.float32),
            pltpu.VMEM((_D, _BPW + 1), jnp.float32),
            pltpu.VMEM((_D, _BPW + 1), jnp.float32),
            pltpu.SemaphoreType.DMA,
            pltpu.SemaphoreType.DMA,
            pltpu.SemaphoreType.DMA,
            pltpu.SemaphoreType.DMA,
        ],
    )
    def k(table_hbm, xT_hbm, out_hbm, idx_all, rows0, rows1, til0, til1,
          gs0, gs1, ss0, ss1):
        rows = (rows0, rows1)
        tils = (til0, til1)
        gsems = (gs0, gs1)
        ssems = (ss0, ss1)

        wid = lax.axis_index("s") * _NC + lax.axis_index("c")
        b0 = wid * _BPW

        def gather_start(h, b):
            pltpu.async_copy(table_hbm.at[idx_all.at[h]], rows[b], gsems[b])

        def gather_wait(h, b):
            pltpu.make_async_copy(table_hbm.at[idx_all.at[h]], rows[b],
                                  gsems[b]).wait()

        def store_start(h, b):
            for r in range(8):
                pltpu.async_copy(tils[b].at[pl.ds(r * 8, 8), pl.ds(0, _BPW)],
                                 out_hbm.at[h, r, wid], ssems[b])

        def store_wait(b):
            for r in range(8):
                pltpu.make_async_copy(tils[b].at[pl.ds(r * 8, 8),
                                                 pl.ds(0, _BPW)],
                                      out_hbm.at[0, r, wid], ssems[b]).wait()

        lanes = lax.iota(jnp.int32, 16)
        dvecs = [lanes + kk * 16 for kk in range(_D // 16)]

        def transpose(b):
            @plsc.parallel_loop(0, _BPW, step=8, unroll=2)
            def _(i):
                for u in range(8):
                    ii = i + u
                    ivec = jnp.full((16,), ii, jnp.int32)
                    for kk in range(_D // 16):
                        v = rows[b][ii, pl.ds(kk * 16, 16)]
                        plsc.store_scatter(tils[b], [dvecs[kk], ivec], v)

        pltpu.sync_copy(xT_hbm.at[:, pl.ds(b0, _BPW)], idx_all)

        gather_start(0, 0)
        gather_wait(0, 0)
        gather_start(1, 1)
        transpose(0)
        store_start(0, 0)
        gather_wait(1, 1)
        gather_start(2, 0)
        transpose(1)
        store_start(1, 1)

        def group(g, carry):
            for b in range(2):
                h = g * 2 + b
                gather_wait(h, b)

                @pl.when(h + 1 < _H)
                def _():
                    gather_start(h + 1, (b + 1) % 2)

                store_wait(b)
                transpose(b)
                store_start(h, b)
            return carry

        lax.fori_loop(1, _H // 2, group, 0)

        store_wait(0)
        store_wait(1)

    return k(table, xT)


def kernel(x, table):
    xT = jnp.transpose(x)
    o = _emb_lookup(table, xT)
    return jnp.transpose(o, (2, 4, 0, 1, 3)).reshape(_B, _H, _D)
